# baseline (speedup 1.0000x reference)
import jax
import jax.numpy as jnp
from jax.experimental import pallas as pl
from jax.experimental.pallas import tpu as pltpu

N = 4096
R = 512
GRID = N // R
F32 = jnp.float32
BF16 = jnp.bfloat16
TDIMS = (((0,), (0,)), ((), ()))


def _p1(adj_ref, x_ref, w0_ref, adj8_ref, d1_ref, cs_ref, v1_ref):
    blk = adj_ref[...]
    adj8_ref[...] = blk.astype(jnp.int8)
    rs = jnp.sum(blk, axis=1, keepdims=True)
    d1 = (rs + 1.0) ** -0.5
    d1_ref[...] = d1

    @pl.when(pl.program_id(0) == 0)
    def _():
        cs_ref[...] = jnp.zeros_like(cs_ref)

    cs_ref[...] += jnp.sum(blk, axis=0, keepdims=True)
    xw = jnp.dot(x_ref[...], w0_ref[...], preferred_element_type=F32)
    v1_ref[...] = (d1 * xw).astype(BF16)


def _p2(adj_ref, v1b_ref, v1f_ref, d1f_ref, b0_ref, w1_ref, v2_ref, acc_ref):
    k = pl.program_id(0)

    @pl.when(k == 0)
    def _():
        acc_ref[...] = jnp.zeros_like(acc_ref)

    acc_ref[...] += jnp.dot(adj_ref[...].astype(BF16), v1b_ref[...],
                            preferred_element_type=F32)

    @pl.when(k == GRID - 1)
    def _():
        pre = acc_ref[...] + v1f_ref[...].astype(F32)
        h1 = jax.nn.relu(d1f_ref[...] * pre + b0_ref[...])
        v2_ref[...] = d1f_ref[...] * jnp.dot(h1, w1_ref[...],
                                             preferred_element_type=F32)


def _p3(adj_ref, v2f_ref, v2b_ref, d1_ref, d2_ref, b1_ref, wg1_ref,
        logits_ref, v3_ref, acc4_ref):
    blk = adj_ref[...].astype(F32)
    acc = jnp.dot(blk, v2f_ref[...], preferred_element_type=F32)
    logits = d1_ref[...] * (acc + v2b_ref[...]) + b1_ref[...]
    logits_ref[...] = logits
    v3 = d2_ref[...] * jnp.dot(logits, wg1_ref[...],
                               preferred_element_type=F32)
    v3_ref[...] = v3

    @pl.when(pl.program_id(0) == 0)
    def _():
        acc4_ref[...] = jnp.zeros_like(acc4_ref)

    acc4_ref[...] += jax.lax.dot_general(blk, v3, TDIMS,
                                         preferred_element_type=F32)


def _p4(adj_ref, acc4b_ref, v3b_ref, d2_ref, bg1_ref, wg2_ref,
        v4_ref, acc5_ref):
    t = jax.nn.relu(d2_ref[...] * (acc4b_ref[...] + v3b_ref[...])
                    + bg1_ref[...])
    v4 = d2_ref[...] * jnp.dot(t, wg2_ref[...], preferred_element_type=F32)
    v4_ref[...] = v4

    @pl.when(pl.program_id(0) == 0)
    def _():
        acc5_ref[...] = jnp.zeros_like(acc5_ref)

    acc5_ref[...] += jax.lax.dot_general(adj_ref[...].astype(F32), v4, TDIMS,
                                         preferred_element_type=F32)


def _p5(acc5_ref, v4f_ref, d2f_ref, bg2_ref, logitsf_ref, out_ref):
    t2 = d2f_ref[...] * (acc5_ref[...] + v4f_ref[...]) + bg2_ref[...]
    t3 = jnp.log(jnp.exp(t2) + 1.1)
    o = logitsf_ref[...] * t3
    m = jnp.max(o, axis=1, keepdims=True)
    lse = m + jnp.log(jnp.sum(jnp.exp(o - m), axis=1, keepdims=True))
    out_ref[...] = o - lse


def _row_blk(f):
    return pl.BlockSpec((R, f), lambda i: (i, 0))


def _full(n, f):
    return pl.BlockSpec((n, f), lambda i: (0, 0))


@jax.jit
def kernel(x, adj, W0, b0, W1, b1, Wg1, bg1, Wg2, bg2):
    D = x.shape[1]
    H = W0.shape[1]
    C = W1.shape[1]
    b0r, b1r = b0[None, :], b1[None, :]
    bg1r, bg2r = bg1[None, :], bg2[None, :]

    adj8, d1, cs, v1 = pl.pallas_call(
        _p1,
        grid=(GRID,),
        in_specs=[_row_blk(N), _row_blk(D), _full(D, H)],
        out_specs=[_row_blk(N), _row_blk(1), _full(1, N), _row_blk(H)],
        out_shape=[jax.ShapeDtypeStruct((N, N), jnp.int8),
                   jax.ShapeDtypeStruct((N, 1), F32),
                   jax.ShapeDtypeStruct((1, N), F32),
                   jax.ShapeDtypeStruct((N, H), BF16)],
    )(adj, x, W0)

    d2 = (cs.reshape(N, 1) + 1.0) ** -0.5

    col_strip = pl.BlockSpec((N, R), lambda k: (0, k))

    v2 = pl.pallas_call(
        _p2,
        grid=(GRID,),
        in_specs=[col_strip, _row_blk(H), _full(N, H), _full(N, 1),
                  _full(1, H), _full(H, C)],
        out_specs=_full(N, C),
        out_shape=jax.ShapeDtypeStruct((N, C), F32),
        scratch_shapes=[pltpu.VMEM((N, H), F32)],
    )(adj8, v1, v1, d1, b0r, W1)

    logits, v3, acc4 = pl.pallas_call(
        _p3,
        grid=(GRID,),
        in_specs=[_row_blk(N), _full(N, C), _row_blk(C), _row_blk(1),
                  _row_blk(1), _full(1, C), _full(C, C)],
        out_specs=[_row_blk(C), _row_blk(C), _full(N, C)],
        out_shape=[jax.ShapeDtypeStruct((N, C), F32),
                   jax.ShapeDtypeStruct((N, C), F32),
                   jax.ShapeDtypeStruct((N, C), F32)],
    )(adj8, v2, v2, d1, d2, b1r, Wg1)

    v4, acc5 = pl.pallas_call(
        _p4,
        grid=(GRID,),
        in_specs=[_row_blk(N), _row_blk(C), _row_blk(C), _row_blk(1),
                  _full(1, C), _full(C, C)],
        out_specs=[_row_blk(C), _full(N, C)],
        out_shape=[jax.ShapeDtypeStruct((N, C), F32),
                   jax.ShapeDtypeStruct((N, C), F32)],
    )(adj8, acc4, v3, d2, bg1r, Wg2)

    out = pl.pallas_call(
        _p5,
        grid=(1,),
        in_specs=[_full(N, C), _full(N, C), _full(N, 1), _full(1, C),
                  _full(N, C)],
        out_specs=_full(N, C),
        out_shape=jax.ShapeDtypeStruct((N, C), F32),
    )(acc5, v4, d2, bg2r, logits)

    return out
